# Initial kernel scaffold; baseline (speedup 1.0000x reference)
import jax
import jax.numpy as jnp
from jax.experimental import pallas as pl


def kernel(x, w1, b1, w2, b2):
    raise NotImplementedError("write your pallas kernel here")


# speedup vs baseline: 2.2883x; 2.2883x over previous
import jax
import jax.numpy as jnp
from jax.experimental import pallas as pl
from jax.experimental.pallas import tpu as pltpu


def _mlp_kernel(x_ref, w1t_ref, b1_ref, w2t_ref, b2_ref, out_ref):
    x = x_ref[...].astype(jnp.bfloat16)
    h = jnp.dot(x, w1t_ref[...], preferred_element_type=jnp.float32)
    h = jnp.maximum(h + b1_ref[...], 0.0).astype(jnp.bfloat16)
    o = jnp.dot(h, w2t_ref[...], preferred_element_type=jnp.float32)
    out_ref[...] = o + b2_ref[...]


def kernel(x, w1, b1, w2, b2):
    B, F = x.shape
    H = w1.shape[0]
    O = w2.shape[0]

    w1t = w1.T.astype(jnp.bfloat16)
    w2t = w2.T.astype(jnp.bfloat16)
    b1r = b1.reshape(1, H)
    b2r = b2.reshape(1, O)

    tb = 1024
    return pl.pallas_call(
        _mlp_kernel,
        out_shape=jax.ShapeDtypeStruct((B, O), jnp.float32),
        grid=(pl.cdiv(B, tb),),
        in_specs=[
            pl.BlockSpec((tb, F), lambda i: (i, 0)),
            pl.BlockSpec((F, H), lambda i: (0, 0)),
            pl.BlockSpec((1, H), lambda i: (0, 0)),
            pl.BlockSpec((H, O), lambda i: (0, 0)),
            pl.BlockSpec((1, O), lambda i: (0, 0)),
        ],
        out_specs=pl.BlockSpec((tb, O), lambda i: (i, 0)),
        compiler_params=pltpu.CompilerParams(
            dimension_semantics=("parallel",),
        ),
        cost_estimate=pl.CostEstimate(
            flops=2 * B * (F * H + H * O),
            transcendentals=0,
            bytes_accessed=4 * (B * F + B * O) + 2 * (F * H + H * O),
        ),
    )(x, w1t, b1r, w2t, b2r)


# speedup vs baseline: 2.5680x; 1.1222x over previous
import jax
import jax.numpy as jnp
from jax.experimental import pallas as pl
from jax.experimental.pallas import tpu as pltpu

_DN_T = (((1,), (1,)), ((), ()))


def _mlp_kernel(x_ref, w1_ref, b1_ref, w2_ref, b2_ref, out_ref):
    h = jax.lax.dot_general(x_ref[...], w1_ref[...], _DN_T,
                            preferred_element_type=jnp.float32)
    h = jnp.maximum(h + b1_ref[...], 0.0)
    o = jax.lax.dot_general(h, w2_ref[...], _DN_T,
                            preferred_element_type=jnp.float32)
    out_ref[...] = o + b2_ref[...]


def kernel(x, w1, b1, w2, b2):
    B, F = x.shape
    H = w1.shape[0]
    O = w2.shape[0]

    b1r = b1.reshape(1, H)
    b2r = b2.reshape(1, O)

    tb = 1024
    return pl.pallas_call(
        _mlp_kernel,
        out_shape=jax.ShapeDtypeStruct((B, O), jnp.float32),
        grid=(pl.cdiv(B, tb),),
        in_specs=[
            pl.BlockSpec((tb, F), lambda i: (i, 0)),
            pl.BlockSpec((H, F), lambda i: (0, 0)),
            pl.BlockSpec((1, H), lambda i: (0, 0)),
            pl.BlockSpec((O, H), lambda i: (0, 0)),
            pl.BlockSpec((1, O), lambda i: (0, 0)),
        ],
        out_specs=pl.BlockSpec((tb, O), lambda i: (i, 0)),
        compiler_params=pltpu.CompilerParams(
            dimension_semantics=("arbitrary",),
        ),
        cost_estimate=pl.CostEstimate(
            flops=2 * B * (F * H + H * O),
            transcendentals=0,
            bytes_accessed=4 * (B * F + B * O + F * H + H * O),
        ),
    )(x, w1, b1r, w2, b2r)
